# baseline (speedup 1.0000x reference)
import functools

import jax
import jax.numpy as jnp
import numpy as np
from jax import lax
from jax.experimental import pallas as pl
from jax.experimental.pallas import tpu as pltpu
from jax.experimental.pallas import tpu_sc as plsc

_N, _E, _DIN, _HID, _LAT, _PTS = 10000, 160000, 384, 128, 64, 1280
_NC, _NS = 2, 16
_NW = _NC * _NS
_CHUNK = 128
_CPT = 40
_NSPLIT = 2
_EPAD = _NW * _CHUNK * _CPT
_EPT = _CPT * _CHUNK
_NACC = 10240
_RPT = _NACC // _NS
_DEGP = _NACC
_BN = 1000
_NBUF = 2

_F32 = jnp.float32

def _eps():
    return jax.random.normal(jax.random.key(42), (_N, _LAT), _F32)

_sc_mesh = plsc.VectorSubcoreMesh(core_axis_name="c", subcore_axis_name="s")


_EREAL = _E // _NW


def _deg_body(ei_hbm, out_hbm, rowp_hbm, colp_hbm, rowv, colv, deg):
    cid = lax.axis_index("c")
    sid = lax.axis_index("s")
    wid = sid * _NC + cid

    padr = jnp.zeros((16,), jnp.int32)
    padc = jnp.full((16,), _N, jnp.int32)

    def pbody(i, c):
        rowv[pl.ds(4992 + i * 16, 16)] = padr
        colv[pl.ds(4992 + i * 16, 16)] = padc
        return c

    lax.fori_loop(0, (_EPT - 4992) // 16, pbody, 0)
    pltpu.sync_copy(ei_hbm.at[pl.ds(wid * _EREAL, _EREAL)],
                    rowv.at[pl.ds(0, _EREAL)])
    pltpu.sync_copy(ei_hbm.at[pl.ds(_E + wid * _EREAL, _EREAL)],
                    colv.at[pl.ds(0, _EREAL)])

    zeros16 = jnp.zeros((16,), _F32)
    ones16 = jnp.ones((16,), _F32)

    def zbody(i, c):
        deg[pl.ds(i * 16, 16)] = zeros16
        return c

    lax.fori_loop(0, _DEGP // 16, zbody, 0)

    def hbody(i, c):
        idx = colv[pl.ds(i * 16, 16)]
        plsc.addupdate_scatter(deg, [idx], ones16)
        return c

    lax.fori_loop(0, _EPT // 16, hbody, 0)
    pltpu.sync_copy(deg, out_hbm.at[wid])
    pltpu.sync_copy(rowv, rowp_hbm.at[pl.ds(wid * _EPT, _EPT)])
    pltpu.sync_copy(colv, colp_hbm.at[pl.ds(wid * _EPT, _EPT)])


_deg_kernel = pl.kernel(
    _deg_body,
    out_type=[
        jax.ShapeDtypeStruct((_NW, _DEGP), _F32),
        jax.ShapeDtypeStruct((_EPAD,), jnp.int32),
        jax.ShapeDtypeStruct((_EPAD,), jnp.int32),
    ],
    mesh=_sc_mesh,
    compiler_params=pltpu.CompilerParams(needs_layout_passes=False),
    scratch_types=[
        pltpu.VMEM((_EPT,), jnp.int32),
        pltpu.VMEM((_EPT,), jnp.int32),
        pltpu.VMEM((_DEGP,), _F32),
    ],
)


def _agg_body(y_hbm, rows_hbm, cols_hbm, out_hbm, rowv, colv, msg, acc,
              *gsems):
    cid = lax.axis_index("c")
    sid = lax.axis_index("s")
    wid = sid * _NC + cid
    pltpu.sync_copy(rows_hbm.at[pl.ds(wid * _CPT, _CPT)], rowv)
    pltpu.sync_copy(cols_hbm.at[pl.ds(wid * _CPT, _CPT)], colv)

    zeros16 = jnp.zeros((16,), _F32)

    def zb(r, c):
        for k in range(_HID // 16):
            msg[0, r, pl.ds(k * 16, 16)] = zeros16
        return c

    lax.fori_loop(0, _CHUNK, zb, 0)

    def zc(k, c):
        pltpu.sync_copy(
            msg.at[0].at[pl.ds(0, min(_CHUNK, 128))],
            acc.at[pl.ds(sid * _RPT + k * min(_CHUNK, 128),
                         min(_CHUNK, 128))])
        return c

    lax.fori_loop(0, _RPT // min(_CHUNK, 128), zc, 0)
    plsc.subcore_barrier()

    _G = _CHUNK // _NSPLIT

    def start_gather(j, b):
        for s in range(_NSPLIT):
            pltpu.async_copy(
                y_hbm.at[rowv.at[j].at[pl.ds(s * _G, _G)]],
                msg.at[b].at[pl.ds(s * _G, _G)],
                gsems[b * _NSPLIT + s])

    def wait_gather(b):
        for s in range(_NSPLIT):
            pltpu.make_async_copy(
                y_hbm.at[pl.ds(0, _G)], msg.at[b].at[pl.ds(s * _G, _G)],
                gsems[b * _NSPLIT + s]).wait()

    for b in range(_NBUF):
        start_gather(b, b)

    def eb(i, c):
        j0 = i * _NBUF
        for b in range(_NBUF):
            wait_gather(b)
            pltpu.sync_copy(msg.at[b], acc.at[colv.at[j0 + b]], add=True)

            @pl.when(i < _CPT // _NBUF - 1)
            def _():
                start_gather(j0 + b + _NBUF, b)

        return c

    lax.fori_loop(0, _CPT // _NBUF, eb, 0)
    plsc.subcore_barrier()
    pltpu.sync_copy(
        acc.at[pl.ds(sid * _RPT, _RPT)],
        out_hbm.at[cid, pl.ds(sid * _RPT, _RPT)],
    )


_agg_kernel = pl.kernel(
    _agg_body,
    out_type=jax.ShapeDtypeStruct((_NC, _NACC, _HID), _F32),
    mesh=_sc_mesh,
    scratch_types=[
        pltpu.VMEM((_CPT, _CHUNK), jnp.int32),
        pltpu.VMEM((_CPT, _CHUNK), jnp.int32),
        pltpu.VMEM((_NBUF, _CHUNK, _HID), _F32),
        pltpu.VMEM_SHARED((_NACC, _HID), _F32),
    ] + [pltpu.SemaphoreType.DMA] * (_NBUF * _NSPLIT),
)


def _dis_body(degp_ref, dis_ref):
    ones = jnp.ones((_NW, 1), _F32)
    deg = lax.dot_general(degp_ref[...], ones,
                          (((0,), (0,)), ((), ())),
                          preferred_element_type=_F32) + 1.0
    dis_ref[...] = lax.rsqrt(deg)


_dis_kernel = pl.pallas_call(
    _dis_body,
    grid=(8,),
    in_specs=[pl.BlockSpec((_NW, _DEGP // 8), lambda i: (0, i))],
    out_specs=pl.BlockSpec((_DEGP // 8, 1), lambda i: (i, 0)),
    out_shape=jax.ShapeDtypeStruct((_DEGP, 1), _F32),
)


def _enc1_body(dis_ref, x_ref, w_ref, y_ref):
    xw = jnp.dot(x_ref[...], w_ref[...], preferred_element_type=_F32)
    y_ref[...] = dis_ref[...] * xw


_enc1_kernel = pl.pallas_call(
    _enc1_body,
    grid=(_N // _BN,),
    in_specs=[
        pl.BlockSpec((_BN, 1), lambda i: (i, 0)),
        pl.BlockSpec((_BN, _DIN), lambda i: (i, 0)),
        pl.BlockSpec((_DIN, _HID), lambda i: (0, 0)),
    ],
    out_specs=pl.BlockSpec((_BN, _HID), lambda i: (i, 0)),
    out_shape=jax.ShapeDtypeStruct((_N, _HID), _F32),
)


def _enc2_body(p_ref, y1_ref, dis_ref, b1_ref, w2_ref, y2_ref):
    dis = dis_ref[...]
    agg = p_ref[0] + p_ref[1] + y1_ref[...]
    h = jnp.maximum(dis * agg + b1_ref[...], 0.0)
    y2_ref[...] = dis * jnp.dot(h, w2_ref[...], preferred_element_type=_F32)


_enc2_kernel = pl.pallas_call(
    _enc2_body,
    grid=(_N // _BN,),
    in_specs=[
        pl.BlockSpec((_NC, _BN, _HID), lambda i: (0, i, 0)),
        pl.BlockSpec((_BN, _HID), lambda i: (i, 0)),
        pl.BlockSpec((_BN, 1), lambda i: (i, 0)),
        pl.BlockSpec((1, _HID), lambda i: (0, 0)),
        pl.BlockSpec((_HID, _HID), lambda i: (0, 0)),
    ],
    out_specs=pl.BlockSpec((_BN, _HID), lambda i: (i, 0)),
    out_shape=jax.ShapeDtypeStruct((_N, _HID), _F32),
)


def _dec_body(p_ref, y2_ref, dis_ref, b2_ref, eps_ref, w3_ref, b3_ref,
              w4_ref, b4_ref, rec_ref, mean_ref, logvar_ref):
    dis = dis_ref[...]
    h2 = dis * (p_ref[0] + p_ref[1] + y2_ref[...]) + b2_ref[...]
    mean = h2[:, :_LAT]
    logvar = h2[:, _LAT:]
    std = jnp.exp(0.5 * logvar)
    z = mean + eps_ref[...] * std
    h3 = jnp.maximum(
        jnp.dot(z, w3_ref[...], preferred_element_type=_F32) + b3_ref[...], 0.0
    )
    for c in range(3):
        rec_ref[c] = (
            jnp.dot(h3, w4_ref[c], preferred_element_type=_F32) + b4_ref[c]
        )
    mean_ref[...] = mean
    logvar_ref[...] = logvar


_dec_kernel = pl.pallas_call(
    _dec_body,
    grid=(_N // _BN,),
    in_specs=[
        pl.BlockSpec((_NC, _BN, _HID), lambda i: (0, i, 0)),
        pl.BlockSpec((_BN, _HID), lambda i: (i, 0)),
        pl.BlockSpec((_BN, 1), lambda i: (i, 0)),
        pl.BlockSpec((1, _HID), lambda i: (0, 0)),
        pl.BlockSpec((_BN, _LAT), lambda i: (i, 0)),
        pl.BlockSpec((_LAT, _HID), lambda i: (0, 0)),
        pl.BlockSpec((1, _HID), lambda i: (0, 0)),
        pl.BlockSpec((3, _HID, _DIN // 3), lambda i: (0, 0, 0)),
        pl.BlockSpec((3, 1, _DIN // 3), lambda i: (0, 0, 0)),
    ],
    out_specs=[
        pl.BlockSpec((3, _BN, _DIN // 3), lambda i: (0, i, 0)),
        pl.BlockSpec((_BN, _LAT), lambda i: (i, 0)),
        pl.BlockSpec((_BN, _LAT), lambda i: (i, 0)),
    ],
    out_shape=[
        jax.ShapeDtypeStruct((3, _N, _DIN // 3), _F32),
        jax.ShapeDtypeStruct((_N, _LAT), _F32),
        jax.ShapeDtypeStruct((_N, _LAT), _F32),
    ],
)


def kernel(x, edge_index, conv1_W, conv1_b, conv2_W, conv2_b,
           fc1_W, fc1_b, fc2_W, fc2_b):
    deg_parts, rowf, colf = _deg_kernel(edge_index.reshape(2 * _E))
    rowp = rowf.reshape(_NW * _CPT, _CHUNK)
    colp = colf.reshape(_NW * _CPT, _CHUNK)

    dis = _dis_kernel(deg_parts)
    y1 = _enc1_kernel(dis, x, conv1_W)
    p1 = _agg_kernel(y1, rowp, colp)
    y2 = _enc2_kernel(p1, y1, dis, conv1_b.reshape(1, _HID), conv2_W)
    p2 = _agg_kernel(y2, rowp, colp)
    w4s = jnp.stack([fc2_W[:, c::3] for c in range(3)])
    b4s = jnp.stack([fc2_b[c::3].reshape(1, -1) for c in range(3)])
    rec2, mean, logvar = _dec_kernel(
        p2, y2, dis, conv2_b.reshape(1, _HID), _eps(),
        fc1_W, fc1_b.reshape(1, _HID), w4s, b4s)
    recon = jnp.transpose(rec2, (1, 2, 0)).reshape(_N // 10, _PTS, 3)
    return recon, mean, logvar


# speedup vs baseline: 1.0341x; 1.0341x over previous
import functools

import jax
import jax.numpy as jnp
import numpy as np
from jax import lax
from jax.experimental import pallas as pl
from jax.experimental.pallas import tpu as pltpu
from jax.experimental.pallas import tpu_sc as plsc

_N, _E, _DIN, _HID, _LAT, _PTS = 10000, 160000, 384, 128, 64, 1280
_NC, _NS = 2, 16
_NW = _NC * _NS
_CHUNK = 128
_CPT = 40
_NSPLIT = 2
_EPAD = _NW * _CHUNK * _CPT
_EPT = _CPT * _CHUNK
_NACC = 10240
_RPT = _NACC // _NS
_DEGP = _NACC
_BN = 1000
_NBUF = 2

_F32 = jnp.float32

def _eps():
    return jax.random.normal(jax.random.key(42), (_N, _LAT), _F32)

_sc_mesh = plsc.VectorSubcoreMesh(core_axis_name="c", subcore_axis_name="s")


_EREAL = _E // _NW


def _deg_body(ei_hbm, out_hbm, rowp_hbm, colp_hbm, rowv, colv, deg):
    cid = lax.axis_index("c")
    sid = lax.axis_index("s")
    wid = sid * _NC + cid

    padr = jnp.zeros((16,), jnp.int32)
    padc = jnp.full((16,), _N, jnp.int32)

    def pbody(i, c):
        rowv[pl.ds(4992 + i * 16, 16)] = padr
        colv[pl.ds(4992 + i * 16, 16)] = padc
        return c

    lax.fori_loop(0, (_EPT - 4992) // 16, pbody, 0)
    pltpu.sync_copy(ei_hbm.at[pl.ds(wid * _EREAL, _EREAL)],
                    rowv.at[pl.ds(0, _EREAL)])
    pltpu.sync_copy(ei_hbm.at[pl.ds(_E + wid * _EREAL, _EREAL)],
                    colv.at[pl.ds(0, _EREAL)])

    zeros16 = jnp.zeros((16,), _F32)
    ones16 = jnp.ones((16,), _F32)

    def zbody(i, c):
        deg[pl.ds(i * 16, 16)] = zeros16
        return c

    lax.fori_loop(0, _DEGP // 16, zbody, 0)

    def hbody(i, c):
        idx = colv[pl.ds(i * 16, 16)]
        plsc.addupdate_scatter(deg, [idx], ones16)
        return c

    lax.fori_loop(0, _EPT // 16, hbody, 0)
    pltpu.sync_copy(deg, out_hbm.at[wid])
    pltpu.sync_copy(rowv, rowp_hbm.at[pl.ds(wid * _EPT, _EPT)])
    pltpu.sync_copy(colv, colp_hbm.at[pl.ds(wid * _EPT, _EPT)])


_deg_kernel = pl.kernel(
    _deg_body,
    out_type=[
        jax.ShapeDtypeStruct((_NW, _DEGP), _F32),
        jax.ShapeDtypeStruct((_EPAD,), jnp.int32),
        jax.ShapeDtypeStruct((_EPAD,), jnp.int32),
    ],
    mesh=_sc_mesh,
    compiler_params=pltpu.CompilerParams(needs_layout_passes=False),
    scratch_types=[
        pltpu.VMEM((_EPT,), jnp.int32),
        pltpu.VMEM((_EPT,), jnp.int32),
        pltpu.VMEM((_DEGP,), _F32),
    ],
)


def _agg_body(y_hbm, rows_hbm, cols_hbm, out_hbm, rowv, colv, msg, acc,
              *gsems):
    cid = lax.axis_index("c")
    sid = lax.axis_index("s")
    wid = sid * _NC + cid
    pltpu.sync_copy(rows_hbm.at[pl.ds(wid * _CPT, _CPT)], rowv)
    pltpu.sync_copy(cols_hbm.at[pl.ds(wid * _CPT, _CPT)], colv)

    zeros16 = jnp.zeros((16,), _F32)

    def zb(r, c):
        for k in range(_HID // 16):
            msg[0, r, pl.ds(k * 16, 16)] = zeros16
        return c

    lax.fori_loop(0, _CHUNK, zb, 0)

    def zc(k, c):
        pltpu.sync_copy(
            msg.at[0].at[pl.ds(0, min(_CHUNK, 128))],
            acc.at[pl.ds(sid * _RPT + k * min(_CHUNK, 128),
                         min(_CHUNK, 128))])
        return c

    lax.fori_loop(0, _RPT // min(_CHUNK, 128), zc, 0)
    plsc.subcore_barrier()

    _G = _CHUNK // _NSPLIT

    def start_gather(j, b):
        for s in range(_NSPLIT):
            pltpu.async_copy(
                y_hbm.at[rowv.at[j].at[pl.ds(s * _G, _G)]],
                msg.at[b].at[pl.ds(s * _G, _G)],
                gsems[b * _NSPLIT + s])

    def wait_gather(b):
        for s in range(_NSPLIT):
            pltpu.make_async_copy(
                y_hbm.at[pl.ds(0, _G)], msg.at[b].at[pl.ds(s * _G, _G)],
                gsems[b * _NSPLIT + s]).wait()

    for b in range(_NBUF):
        start_gather(b, b)

    def eb(i, c):
        j0 = i * _NBUF
        for b in range(_NBUF):
            wait_gather(b)
            pltpu.sync_copy(msg.at[b], acc.at[colv.at[j0 + b]], add=True)

            @pl.when(i < _CPT // _NBUF - 1)
            def _():
                start_gather(j0 + b + _NBUF, b)

        return c

    lax.fori_loop(0, _CPT // _NBUF, eb, 0)
    plsc.subcore_barrier()
    pltpu.sync_copy(
        acc.at[pl.ds(sid * _RPT, _RPT)],
        out_hbm.at[cid, pl.ds(sid * _RPT, _RPT)],
    )


_agg_kernel = pl.kernel(
    _agg_body,
    out_type=jax.ShapeDtypeStruct((_NC, _NACC, _HID), _F32),
    mesh=_sc_mesh,
    scratch_types=[
        pltpu.VMEM((_CPT, _CHUNK), jnp.int32),
        pltpu.VMEM((_CPT, _CHUNK), jnp.int32),
        pltpu.VMEM((_NBUF, _CHUNK, _HID), _F32),
        pltpu.VMEM_SHARED((_NACC, _HID), _F32),
    ] + [pltpu.SemaphoreType.DMA] * (_NBUF * _NSPLIT),
)


def _dis_body(degp_ref, dis_ref):
    ones = jnp.ones((_NW, 1), _F32)
    deg = lax.dot_general(degp_ref[...], ones,
                          (((0,), (0,)), ((), ())),
                          preferred_element_type=_F32) + 1.0
    dis_ref[...] = lax.rsqrt(deg)


_dis_kernel = pl.pallas_call(
    _dis_body,
    grid=(8,),
    in_specs=[pl.BlockSpec((_NW, _DEGP // 8), lambda i: (0, i))],
    out_specs=pl.BlockSpec((_DEGP // 8, 1), lambda i: (i, 0)),
    out_shape=jax.ShapeDtypeStruct((_DEGP, 1), _F32),
)


def _enc1_body(dis_ref, x_ref, w_ref, y_ref):
    xw = jnp.dot(x_ref[...], w_ref[...], preferred_element_type=_F32)
    y_ref[...] = dis_ref[...] * xw


_enc1_kernel = pl.pallas_call(
    _enc1_body,
    grid=(_N // _BN,),
    in_specs=[
        pl.BlockSpec((_BN, 1), lambda i: (i, 0)),
        pl.BlockSpec((_BN, _DIN), lambda i: (i, 0)),
        pl.BlockSpec((_DIN, _HID), lambda i: (0, 0)),
    ],
    out_specs=pl.BlockSpec((_BN, _HID), lambda i: (i, 0)),
    out_shape=jax.ShapeDtypeStruct((_N, _HID), _F32),
)


def _enc2_body(p_ref, y1_ref, dis_ref, b1_ref, w2_ref, y2_ref):
    dis = dis_ref[...]
    agg = p_ref[0] + p_ref[1] + y1_ref[...]
    h = jnp.maximum(dis * agg + b1_ref[...], 0.0)
    y2_ref[...] = dis * jnp.dot(h, w2_ref[...], preferred_element_type=_F32)


_enc2_kernel = pl.pallas_call(
    _enc2_body,
    grid=(_N // _BN,),
    in_specs=[
        pl.BlockSpec((_NC, _BN, _HID), lambda i: (0, i, 0)),
        pl.BlockSpec((_BN, _HID), lambda i: (i, 0)),
        pl.BlockSpec((_BN, 1), lambda i: (i, 0)),
        pl.BlockSpec((1, _HID), lambda i: (0, 0)),
        pl.BlockSpec((_HID, _HID), lambda i: (0, 0)),
    ],
    out_specs=pl.BlockSpec((_BN, _HID), lambda i: (i, 0)),
    out_shape=jax.ShapeDtypeStruct((_N, _HID), _F32),
)


def _dec_body(p_ref, y2_ref, dis_ref, b2_ref, eps_ref, w3_ref, b3_ref,
              w4_ref, b4_ref, rec_ref, mean_ref, logvar_ref):
    dis = dis_ref[...]
    h2 = dis * (p_ref[0] + p_ref[1] + y2_ref[...]) + b2_ref[...]
    mean = h2[:, :_LAT]
    logvar = h2[:, _LAT:]
    std = jnp.exp(0.5 * logvar)
    z = mean + eps_ref[...] * std
    h3 = jnp.maximum(
        jnp.dot(z, w3_ref[...], preferred_element_type=_F32) + b3_ref[...], 0.0
    )
    for c in range(3):
        rc = jnp.dot(h3, w4_ref[c], preferred_element_type=_F32) + b4_ref[c]
        rec_ref[c] = rc.reshape(_BND // 10, _PTS)
    mean_ref[...] = mean
    logvar_ref[...] = logvar


_BND = 2000

_dec_kernel = pl.pallas_call(
    _dec_body,
    grid=(_N // _BND,),
    in_specs=[
        pl.BlockSpec((_NC, _BND, _HID), lambda i: (0, i, 0)),
        pl.BlockSpec((_BND, _HID), lambda i: (i, 0)),
        pl.BlockSpec((_BND, 1), lambda i: (i, 0)),
        pl.BlockSpec((1, _HID), lambda i: (0, 0)),
        pl.BlockSpec((_BND, _LAT), lambda i: (i, 0)),
        pl.BlockSpec((_LAT, _HID), lambda i: (0, 0)),
        pl.BlockSpec((1, _HID), lambda i: (0, 0)),
        pl.BlockSpec((3, _HID, _DIN // 3), lambda i: (0, 0, 0)),
        pl.BlockSpec((3, 1, _DIN // 3), lambda i: (0, 0, 0)),
    ],
    out_specs=[
        pl.BlockSpec((3, _BND // 10, _PTS), lambda i: (0, i, 0)),
        pl.BlockSpec((_BND, _LAT), lambda i: (i, 0)),
        pl.BlockSpec((_BND, _LAT), lambda i: (i, 0)),
    ],
    out_shape=[
        jax.ShapeDtypeStruct((3, _N // 10, _PTS), _F32),
        jax.ShapeDtypeStruct((_N, _LAT), _F32),
        jax.ShapeDtypeStruct((_N, _LAT), _F32),
    ],
)


def kernel(x, edge_index, conv1_W, conv1_b, conv2_W, conv2_b,
           fc1_W, fc1_b, fc2_W, fc2_b):
    deg_parts, rowf, colf = _deg_kernel(edge_index.reshape(2 * _E))
    rowp = rowf.reshape(_NW * _CPT, _CHUNK)
    colp = colf.reshape(_NW * _CPT, _CHUNK)

    dis = _dis_kernel(deg_parts)
    y1 = _enc1_kernel(dis, x, conv1_W)
    p1 = _agg_kernel(y1, rowp, colp)
    y2 = _enc2_kernel(p1, y1, dis, conv1_b.reshape(1, _HID), conv2_W)
    p2 = _agg_kernel(y2, rowp, colp)
    w4s = jnp.stack([fc2_W[:, c::3] for c in range(3)])
    b4s = jnp.stack([fc2_b[c::3].reshape(1, -1) for c in range(3)])
    rec2, mean, logvar = _dec_kernel(
        p2, y2, dis, conv2_b.reshape(1, _HID), _eps(),
        fc1_W, fc1_b.reshape(1, _HID), w4s, b4s)
    recon = jnp.transpose(rec2, (1, 2, 0))
    return recon, mean, logvar
